# Initial kernel scaffold; baseline (speedup 1.0000x reference)
import jax
import jax.numpy as jnp
from jax.experimental import pallas as pl


def kernel(node_features, edge_features, radial_embedding, senders, receivers, node_species, W_up, Wr1, br1, Wr2, W_down, W_skip):
    raise NotImplementedError("write your pallas kernel here")


# speedup vs baseline: 1.7377x; 1.7377x over previous
import functools
import math

import jax
import jax.numpy as jnp
from jax import lax
from jax.experimental import pallas as pl
from jax.experimental.pallas import tpu as pltpu
from jax.experimental.pallas import tpu_sc as plsc

AVG_NEIGH = 32.0
NUM_SCALARS = 32
NUM_VEC = 32

NC = 2
NS = 16
NW = NC * NS
CHUNK = 128


def _feats_body(nf_ref, wup_ref, out_ref):
    out_ref[...] = jnp.dot(nf_ref[...], wup_ref[...],
                           preferred_element_type=jnp.float32)


def _edge_scalar_body(ef_ref, re_ref, wr1_ref, br1_ref, wr2_ref, out_ref):
    h = jnp.dot(re_ref[...], wr1_ref[...],
                preferred_element_type=jnp.float32) + br1_ref[...]
    h = h * jax.nn.sigmoid(h)
    rw = jnp.dot(h, wr2_ref[...], preferred_element_type=jnp.float32)
    es = jnp.sum(ef_ref[...] * rw, axis=1)
    out_ref[...] = es.reshape(out_ref.shape)


def _make_sc_kernel(n_nodes, d, cpw):
    mesh = plsc.VectorSubcoreMesh(core_axis_name="c", subcore_axis_name="s")
    rows_per_sub = (n_nodes // NS) // 8 * 8
    tail_rows = n_nodes - rows_per_sub * NS
    tail_base = rows_per_sub * NS

    @functools.partial(
        pl.kernel,
        out_type=jax.ShapeDtypeStruct((NC, n_nodes, d), jnp.float32),
        mesh=mesh,
        compiler_params=pltpu.CompilerParams(needs_layout_passes=False),
        scratch_types=[
            pltpu.VMEM((cpw, CHUNK), jnp.int32),
            pltpu.VMEM((cpw, CHUNK), jnp.int32),
            pltpu.VMEM((cpw, CHUNK), jnp.float32),
            pltpu.VMEM((CHUNK, d), jnp.float32),
            pltpu.VMEM_SHARED((n_nodes, d), jnp.float32),
            pltpu.SemaphoreType.DMA,
        ],
    )
    def sc_kernel(feats_hbm, snd_hbm, rcv_hbm, es_hbm, out_hbm,
                  snd_v, rcv_v, es_v, rows_v, agg_sh, sem):
        cid = lax.axis_index("c")
        sid = lax.axis_index("s")

        zero16 = jnp.zeros((16,), jnp.float32)

        def zrow(i, _):
            for dd in range(d // 16):
                rows_v[i, pl.ds(dd * 16, 16)] = zero16
            return 0

        lax.fori_loop(0, CHUNK, zrow, 0)
        base_node = sid * rows_per_sub
        full, rem = divmod(rows_per_sub, CHUNK)
        for k in range(full):
            pltpu.sync_copy(rows_v, agg_sh.at[pl.ds(base_node + k * CHUNK, CHUNK)])
        if rem:
            pltpu.sync_copy(rows_v.at[pl.ds(0, rem)],
                            agg_sh.at[pl.ds(base_node + full * CHUNK, rem)])
        if tail_rows:
            @pl.when(sid == 0)
            def _zero_tail():
                pltpu.sync_copy(rows_v.at[pl.ds(0, tail_rows)],
                                agg_sh.at[pl.ds(tail_base, tail_rows)])
        plsc.subcore_barrier()

        wid = cid * NS + sid
        base_row = wid * cpw
        pltpu.sync_copy(snd_hbm.at[pl.ds(base_row, cpw)], snd_v)
        pltpu.sync_copy(rcv_hbm.at[pl.ds(base_row, cpw)], rcv_v)
        pltpu.sync_copy(es_hbm.at[pl.ds(base_row, cpw)], es_v)

        def chunk_body(c, _):
            pltpu.async_copy(feats_hbm.at[snd_v.at[c]], rows_v, sem).wait()

            es_row = es_v.at[c]

            def edge_body(e, _):
                s = plsc.load_gather(es_row, [jnp.full((16,), e, jnp.int32)])
                for dd in range(d // 16):
                    sl = pl.ds(dd * 16, 16)
                    rows_v[e, sl] = rows_v[e, sl] * s
                return 0

            lax.fori_loop(0, CHUNK, edge_body, 0)
            pltpu.sync_copy(rows_v, agg_sh.at[rcv_v.at[c]], add=True)
            return 0

        lax.fori_loop(0, cpw, chunk_body, 0)
        plsc.subcore_barrier()

        pltpu.sync_copy(agg_sh.at[pl.ds(base_node, rows_per_sub)],
                        out_hbm.at[cid, pl.ds(base_node, rows_per_sub)])
        if tail_rows:
            @pl.when(sid == 0)
            def _write_tail():
                pltpu.sync_copy(agg_sh.at[pl.ds(tail_base, tail_rows)],
                                out_hbm.at[cid, pl.ds(tail_base, tail_rows)])

    return sc_kernel


def _final_body(nf_ref, a0_ref, a1_ref, sp_ref, wd_ref, ws_ref, out_ref):
    inv = 1.0 / math.sqrt(AVG_NEIGH)
    agg = (a0_ref[...] + a1_ref[...]) * inv
    down = jnp.dot(agg, wd_ref[...], preferred_element_type=jnp.float32)
    nf = nf_ref[...]
    sp = sp_ref[...]
    skip = jnp.zeros_like(down)
    for s in range(ws_ref.shape[0]):
        contrib = jnp.dot(nf, ws_ref[s], preferred_element_type=jnp.float32)
        skip = skip + jnp.where(sp == s, contrib, 0.0)
    x = 0.5 * (down + skip)
    nsc = NUM_SCALARS
    nv3 = 3 * NUM_VEC
    scal = x[:, :nsc]
    vecs = x[:, nsc:nsc + nv3]
    gates = x[:, nsc + nv3:]
    g = gates * jax.nn.sigmoid(gates)
    row_i = lax.broadcasted_iota(jnp.int32, (NUM_VEC, nv3), 0)
    col_i = lax.broadcasted_iota(jnp.int32, (NUM_VEC, nv3), 1)
    rep = (row_i == col_i // 3).astype(jnp.float32)
    gmul = jnp.dot(g, rep, preferred_element_type=jnp.float32)
    out_ref[...] = jnp.concatenate(
        [scal * jax.nn.sigmoid(scal), vecs * gmul], axis=1)


def kernel(node_features, edge_features, radial_embedding, senders, receivers,
           node_species, W_up, Wr1, br1, Wr2, W_down, W_skip):
    n, d = node_features.shape
    e, sh = edge_features.shape
    r = radial_embedding.shape[1]
    gate_d = W_down.shape[1]

    nblk = 2000
    feats = pl.pallas_call(
        _feats_body,
        grid=(n // nblk,),
        in_specs=[pl.BlockSpec((nblk, d), lambda i: (i, 0)),
                  pl.BlockSpec((d, d), lambda i: (0, 0))],
        out_specs=pl.BlockSpec((nblk, d), lambda i: (i, 0)),
        out_shape=jax.ShapeDtypeStruct((n, d), jnp.float32),
    )(node_features, W_up)

    cpw = -(-e // (NW * CHUNK))
    cpw = -(-cpw // 8) * 8
    e_pad = NW * CHUNK * cpw
    pad = e_pad - e
    snd2 = jnp.pad(senders, (0, pad)).reshape(e_pad // CHUNK, CHUNK)
    rcv2 = jnp.pad(receivers, (0, pad)).reshape(e_pad // CHUNK, CHUNK)
    ef_p = jnp.pad(edge_features, ((0, pad), (0, 0)))
    re_p = jnp.pad(radial_embedding, ((0, pad), (0, 0)))

    eblk = 16
    es2 = pl.pallas_call(
        _edge_scalar_body,
        grid=(e_pad // (eblk * CHUNK),),
        in_specs=[pl.BlockSpec((eblk * CHUNK, sh), lambda i: (i, 0)),
                  pl.BlockSpec((eblk * CHUNK, r), lambda i: (i, 0)),
                  pl.BlockSpec((r, r), lambda i: (0, 0)),
                  pl.BlockSpec((1, r), lambda i: (0, 0)),
                  pl.BlockSpec((r, sh), lambda i: (0, 0))],
        out_specs=pl.BlockSpec((eblk, CHUNK), lambda i: (i, 0)),
        out_shape=jax.ShapeDtypeStruct((e_pad // CHUNK, CHUNK), jnp.float32),
    )(ef_p, re_p, Wr1, br1.reshape(1, r), Wr2)

    agg2 = _make_sc_kernel(n, d, cpw)(feats, snd2, rcv2, es2)

    out = pl.pallas_call(
        _final_body,
        grid=(n // nblk,),
        in_specs=[pl.BlockSpec((nblk, d), lambda i: (i, 0)),
                  pl.BlockSpec((nblk, d), lambda i: (i, 0)),
                  pl.BlockSpec((nblk, d), lambda i: (i, 0)),
                  pl.BlockSpec((nblk, 1), lambda i: (i, 0)),
                  pl.BlockSpec((d, gate_d), lambda i: (0, 0)),
                  pl.BlockSpec(W_skip.shape, lambda i: (0, 0, 0))],
        out_specs=pl.BlockSpec((nblk, d), lambda i: (i, 0)),
        out_shape=jax.ShapeDtypeStruct((n, d), jnp.float32),
    )(node_features, agg2[0], agg2[1], node_species.reshape(n, 1),
      W_down, W_skip)
    return out


# speedup vs baseline: 3.1239x; 1.7977x over previous
import functools
import math

import jax
import jax.numpy as jnp
from jax import lax
from jax.experimental import pallas as pl
from jax.experimental.pallas import tpu as pltpu
from jax.experimental.pallas import tpu_sc as plsc

AVG_NEIGH = 32.0
NUM_SCALARS = 32
NUM_VEC = 32

NC = 2
NS = 16
NW = NC * NS
CHUNK = 64


def _feats_body(nf_ref, wup_ref, out_ref):
    out_ref[...] = jnp.dot(nf_ref[...], wup_ref[...],
                           preferred_element_type=jnp.float32)


def _edge_scalar_body(ef_ref, re_ref, wr1_ref, br1_ref, wr2_ref, out_ref):
    h = jnp.dot(re_ref[...], wr1_ref[...],
                preferred_element_type=jnp.float32) + br1_ref[...]
    h = h * jax.nn.sigmoid(h)
    rw = jnp.dot(h, wr2_ref[...], preferred_element_type=jnp.float32)
    es = jnp.sum(ef_ref[...] * rw, axis=1)
    out_ref[...] = es.reshape(out_ref.shape)


def _make_sc_kernel(n_nodes, d, n_edges):
    mesh = plsc.VectorSubcoreMesh(core_axis_name="c", subcore_axis_name="s")
    rows_per_sub = (n_nodes // NS) // 8 * 8
    tail_rows = n_nodes - rows_per_sub * NS
    tail_base = rows_per_sub * NS
    ew = n_edges // NW
    assert ew * NW == n_edges and ew % 8 == 0
    nch, etail = divmod(ew, CHUNK)

    @functools.partial(
        pl.kernel,
        out_type=jax.ShapeDtypeStruct((NC, n_nodes, d), jnp.float32),
        mesh=mesh,
        compiler_params=pltpu.CompilerParams(needs_layout_passes=False),
        scratch_types=[
            pltpu.VMEM((ew,), jnp.int32),
            pltpu.VMEM((ew,), jnp.int32),
            pltpu.VMEM((ew,), jnp.float32),
            pltpu.VMEM((2, CHUNK, d), jnp.float32),
            pltpu.VMEM((2, CHUNK), jnp.int32),
            pltpu.VMEM((etail if etail else 16,), jnp.int32),
            pltpu.VMEM_SHARED((n_nodes, d), jnp.float32),
            pltpu.SemaphoreType.DMA,
            pltpu.SemaphoreType.DMA,
        ],
    )
    def sc_kernel(feats_hbm, snd_hbm, rcv_hbm, es_hbm, out_hbm,
                  snd_v, rcv_v, es_v, rows_v, ridx_v, tidx_v, agg_sh,
                  gsem, msem):
        cid = lax.axis_index("c")
        sid = lax.axis_index("s")
        wid = cid * NS + sid
        ebase = wid * ew

        meta = [
            pltpu.async_copy(snd_hbm.at[pl.ds(ebase, ew)], snd_v, msem),
            pltpu.async_copy(rcv_hbm.at[pl.ds(ebase, ew)], rcv_v, msem),
            pltpu.async_copy(es_hbm.at[pl.ds(ebase, ew)], es_v, msem),
        ]

        zero16 = jnp.zeros((16,), jnp.float32)

        def zrow(i, _):
            for dd in range(d // 16):
                rows_v[0, i, pl.ds(dd * 16, 16)] = zero16
            return 0

        lax.fori_loop(0, CHUNK, zrow, 0)
        base_node = sid * rows_per_sub
        zfull, zrem = divmod(rows_per_sub, CHUNK)
        for k in range(zfull):
            pltpu.sync_copy(rows_v.at[0],
                            agg_sh.at[pl.ds(base_node + k * CHUNK, CHUNK)])
        if zrem:
            pltpu.sync_copy(rows_v.at[0, pl.ds(0, zrem)],
                            agg_sh.at[pl.ds(base_node + zfull * CHUNK, zrem)])
        if tail_rows:
            @pl.when(sid == 0)
            def _zero_tail():
                pltpu.sync_copy(rows_v.at[0, pl.ds(0, tail_rows)],
                                agg_sh.at[pl.ds(tail_base, tail_rows)])
        for m in meta:
            m.wait()
        plsc.subcore_barrier()

        def issue_gather(c, b):
            return pltpu.async_copy(
                feats_hbm.at[snd_v.at[pl.ds(c * CHUNK, CHUNK)]],
                rows_v.at[b], gsem)

        def wait_gather(c, b):
            pltpu.make_async_copy(
                feats_hbm.at[snd_v.at[pl.ds(c * CHUNK, CHUNK)]],
                rows_v.at[b], gsem).wait()

        def scale_rows(c, b, nrows):
            def edge_body(e, _):
                s = plsc.load_gather(
                    es_v, [jnp.full((16,), c * CHUNK + e, jnp.int32)])
                for dd in range(d // 16):
                    sl = pl.ds(dd * 16, 16)
                    rows_v[b, e, sl] = rows_v[b, e, sl] * s
                return 0

            lax.fori_loop(0, nrows, edge_body, 0, unroll=4)

        def scale_and_scatter(c, b):
            scale_rows(c, b, CHUNK)
            for i in range(CHUNK // 16):
                ridx_v[b, pl.ds(16 * i, 16)] = (
                    rcv_v[pl.ds(c * CHUNK + 16 * i, 16)])
            pltpu.sync_copy(rows_v.at[b], agg_sh.at[ridx_v.at[b]], add=True)

        issue_gather(0, 0)

        def chunk_body(c, _):
            b = lax.rem(c, 2)

            @pl.when(c + 1 < nch)
            def _prefetch():
                issue_gather(c + 1, 1 - b)

            wait_gather(c, b)
            scale_and_scatter(c, b)
            return 0

        lax.fori_loop(0, nch, chunk_body, 0)

        if etail:
            pltpu.async_copy(
                feats_hbm.at[snd_v.at[pl.ds(nch * CHUNK, etail)]],
                rows_v.at[0, pl.ds(0, etail)], gsem).wait()
            scale_rows(nch, 0, etail)
            for i in range(etail // 16):
                tidx_v[pl.ds(16 * i, 16)] = (
                    rcv_v[pl.ds(nch * CHUNK + 16 * i, 16)])
            pltpu.sync_copy(rows_v.at[0, pl.ds(0, etail)],
                            agg_sh.at[tidx_v], add=True)

        plsc.subcore_barrier()

        pltpu.sync_copy(agg_sh.at[pl.ds(base_node, rows_per_sub)],
                        out_hbm.at[cid, pl.ds(base_node, rows_per_sub)])
        if tail_rows:
            @pl.when(sid == 0)
            def _write_tail():
                pltpu.sync_copy(agg_sh.at[pl.ds(tail_base, tail_rows)],
                                out_hbm.at[cid, pl.ds(tail_base, tail_rows)])

    return sc_kernel


def _final_body(nf_ref, a0_ref, a1_ref, sp_ref, wd_ref, ws_ref, out_ref):
    inv = 1.0 / math.sqrt(AVG_NEIGH)
    agg = (a0_ref[...] + a1_ref[...]) * inv
    down = jnp.dot(agg, wd_ref[...], preferred_element_type=jnp.float32)
    nf = nf_ref[...]
    sp = sp_ref[...]
    skip = jnp.zeros_like(down)
    for s in range(ws_ref.shape[0]):
        contrib = jnp.dot(nf, ws_ref[s], preferred_element_type=jnp.float32)
        skip = skip + jnp.where(sp == s, contrib, 0.0)
    x = 0.5 * (down + skip)
    nsc = NUM_SCALARS
    nv3 = 3 * NUM_VEC
    scal = x[:, :nsc]
    vecs = x[:, nsc:nsc + nv3]
    gates = x[:, nsc + nv3:]
    g = gates * jax.nn.sigmoid(gates)
    row_i = lax.broadcasted_iota(jnp.int32, (NUM_VEC, nv3), 0)
    col_i = lax.broadcasted_iota(jnp.int32, (NUM_VEC, nv3), 1)
    rep = (row_i == col_i // 3).astype(jnp.float32)
    gmul = jnp.dot(g, rep, preferred_element_type=jnp.float32)
    out_ref[...] = jnp.concatenate(
        [scal * jax.nn.sigmoid(scal), vecs * gmul], axis=1)


def kernel(node_features, edge_features, radial_embedding, senders, receivers,
           node_species, W_up, Wr1, br1, Wr2, W_down, W_skip):
    n, d = node_features.shape
    e, sh = edge_features.shape
    r = radial_embedding.shape[1]
    gate_d = W_down.shape[1]

    nblk = 2000
    feats = pl.pallas_call(
        _feats_body,
        grid=(n // nblk,),
        in_specs=[pl.BlockSpec((nblk, d), lambda i: (i, 0)),
                  pl.BlockSpec((d, d), lambda i: (0, 0))],
        out_specs=pl.BlockSpec((nblk, d), lambda i: (i, 0)),
        out_shape=jax.ShapeDtypeStruct((n, d), jnp.float32),
    )(node_features, W_up)

    eblk = 2560
    es2 = pl.pallas_call(
        _edge_scalar_body,
        grid=(e // eblk,),
        in_specs=[pl.BlockSpec((eblk, sh), lambda i: (i, 0)),
                  pl.BlockSpec((eblk, r), lambda i: (i, 0)),
                  pl.BlockSpec((r, r), lambda i: (0, 0)),
                  pl.BlockSpec((1, r), lambda i: (0, 0)),
                  pl.BlockSpec((r, sh), lambda i: (0, 0))],
        out_specs=pl.BlockSpec((eblk, 1), lambda i: (i, 0)),
        out_shape=jax.ShapeDtypeStruct((e, 1), jnp.float32),
    )(edge_features, radial_embedding, Wr1, br1.reshape(1, r), Wr2)

    agg2 = _make_sc_kernel(n, d, e)(feats, senders, receivers,
                                    es2.reshape(-1))

    out = pl.pallas_call(
        _final_body,
        grid=(n // nblk,),
        in_specs=[pl.BlockSpec((nblk, d), lambda i: (i, 0)),
                  pl.BlockSpec((nblk, d), lambda i: (i, 0)),
                  pl.BlockSpec((nblk, d), lambda i: (i, 0)),
                  pl.BlockSpec((nblk, 1), lambda i: (i, 0)),
                  pl.BlockSpec((d, gate_d), lambda i: (0, 0)),
                  pl.BlockSpec(W_skip.shape, lambda i: (0, 0, 0))],
        out_specs=pl.BlockSpec((nblk, d), lambda i: (i, 0)),
        out_shape=jax.ShapeDtypeStruct((n, d), jnp.float32),
    )(node_features, agg2[0], agg2[1], node_species.reshape(n, 1),
      W_down, W_skip)
    return out
